# baseline (speedup 1.0000x reference)
import functools

import jax
import jax.numpy as jnp
from jax import lax
from jax.experimental import pallas as pl
from jax.experimental.pallas import tpu as pltpu
from jax.experimental.pallas import tpu_sc as plsc

NN = 10000
EE = 320000
DD = 128
HH = 64
NC, NS, LL = 2, 16, 16
NWORK = NC * NS
CHUNK = 128
NCH = 80
NBUF = 2
NGRP = NCH // NBUF
EPAD = NWORK * NCH * CHUNK
ROWS_PER_TILE = 632
NP = NS * ROWS_PER_TILE
EW = EE // NWORK

_MESH = plsc.VectorSubcoreMesh(
    core_axis_name="c", subcore_axis_name="s", num_cores=NC, num_subcores=NS)


def _wid():
  return lax.axis_index("c") * NS + lax.axis_index("s")


@functools.partial(
    pl.kernel,
    out_type=jax.ShapeDtypeStruct((NC, NP, LL), jnp.float32),
    mesh=_MESH,
    compiler_params=pltpu.CompilerParams(use_tc_tiling_on_sc=False),
    scratch_types=[
        pltpu.VMEM((NCH, CHUNK), jnp.int32),
        pltpu.VMEM((CHUNK, LL), jnp.float32),
        pltpu.VMEM_SHARED((NP, LL), jnp.float32),
    ],
)
def _sc_degree(dst3, zeros16, ones16, cnt_out, dst_l, ones_v, acc):
  c = lax.axis_index("c")
  s = lax.axis_index("s")
  w = _wid()
  rbase = s * ROWS_PER_TILE
  pltpu.sync_copy(zeros16.at[pl.ds(rbase, ROWS_PER_TILE)],
                  acc.at[pl.ds(rbase, ROWS_PER_TILE)])
  pltpu.sync_copy(ones16, ones_v)
  pltpu.sync_copy(dst3.at[w], dst_l)
  plsc.subcore_barrier()

  @pl.loop(0, NCH)
  def _(j):
    pltpu.sync_copy(ones_v, acc.at[dst_l.at[j]], add=True)

  plsc.subcore_barrier()
  pltpu.sync_copy(acc.at[pl.ds(rbase, ROWS_PER_TILE)],
                  cnt_out.at[c, pl.ds(rbase, ROWS_PER_TILE)])


@functools.partial(
    pl.kernel,
    out_type=jax.ShapeDtypeStruct((NC, NP, HH), jnp.float32),
    mesh=_MESH,
    compiler_params=pltpu.CompilerParams(use_tc_tiling_on_sc=False),
    scratch_types=[
        pltpu.VMEM((NCH, CHUNK), jnp.int32),
        pltpu.VMEM((NCH, CHUNK), jnp.int32),
        [pltpu.VMEM((CHUNK, HH), jnp.float32) for _ in range(NBUF)],
        [pltpu.SemaphoreType.DMA for _ in range(NBUF)],
        [pltpu.SemaphoreType.DMA for _ in range(NBUF)],
        pltpu.VMEM_SHARED((NP, HH), jnp.float32),
        pltpu.VMEM_SHARED((NP, HH), jnp.float32),
    ],
)
def _sc_msgpass(g_tab, src3, dst3, zeros64, s_out,
                src_l, dst_l, rows, gsem, ssem, acc, g_sp):
  c = lax.axis_index("c")
  s = lax.axis_index("s")
  w = _wid()
  rbase = s * ROWS_PER_TILE
  pltpu.sync_copy(g_tab.at[pl.ds(rbase, ROWS_PER_TILE)],
                  g_sp.at[pl.ds(rbase, ROWS_PER_TILE)])
  pltpu.sync_copy(zeros64.at[pl.ds(rbase, ROWS_PER_TILE)],
                  acc.at[pl.ds(rbase, ROWS_PER_TILE)])
  pltpu.sync_copy(src3.at[w], src_l)
  pltpu.sync_copy(dst3.at[w], dst_l)
  plsc.subcore_barrier()

  for b in range(NBUF):
    pltpu.async_copy(g_sp.at[src_l.at[b]], rows[b], gsem[b])

  @pl.loop(0, NGRP)
  def _(g):
    base = g * NBUF
    for b in range(NBUF):
      pltpu.make_async_copy(g_sp.at[src_l.at[base + b]], rows[b], gsem[b]).wait()
      pltpu.async_copy(rows[b], acc.at[dst_l.at[base + b]], ssem[b], add=True)

    @pl.when(g < NGRP - 1)
    def _():
      for b in range(NBUF):
        pltpu.make_async_copy(rows[b], acc.at[dst_l.at[base + b]], ssem[b]).wait()
        pltpu.async_copy(g_sp.at[src_l.at[base + NBUF + b]], rows[b], gsem[b])

  for b in range(NBUF):
    pltpu.make_async_copy(rows[b], acc.at[dst_l.at[(NGRP - 1) * NBUF + b]],
                          ssem[b]).wait()

  plsc.subcore_barrier()
  pltpu.sync_copy(acc.at[pl.ds(rbase, ROWS_PER_TILE)],
                  s_out.at[c, pl.ds(rbase, ROWS_PER_TILE)])


@functools.partial(
    pl.kernel,
    out_type=jax.ShapeDtypeStruct((NWORK, EW), jnp.float32),
    mesh=_MESH,
    compiler_params=pltpu.CompilerParams(
        use_tc_tiling_on_sc=False, needs_layout_passes=False),
    scratch_types=[
        pltpu.VMEM((NN,), jnp.float32),
        pltpu.VMEM((NN,), jnp.float32),
        pltpu.VMEM((EW,), jnp.int32),
        pltpu.VMEM((EW,), jnp.int32),
        pltpu.VMEM((EW,), jnp.float32),
    ],
)
def _sc_decode(s_tab, t_tab, ei3, dec_out, s_l, t_l, src_l, dst_l, ob):
  w = _wid()
  pltpu.sync_copy(s_tab, s_l)
  pltpu.sync_copy(t_tab, t_l)
  pltpu.sync_copy(ei3.at[0, w], src_l)
  pltpu.sync_copy(ei3.at[1, w], dst_l)

  @pl.loop(0, EW // LL, unroll=8)
  def _(i):
    si = src_l[pl.ds(i * LL, LL)]
    di = dst_l[pl.ds(i * LL, LL)]
    sv = plsc.load_gather(s_l, [si])
    tv = plsc.load_gather(t_l, [di])
    y = sv + tv
    ob[pl.ds(i * LL, LL)] = 1.0 / (1.0 + jnp.exp(-y))

  pltpu.sync_copy(ob, dec_out.at[w])


def _dinv_from_cnt(cnt_ref):
  cnt = cnt_ref[0, 0:NN, 0:1] + cnt_ref[1, 0:NN, 0:1]
  return lax.rsqrt(cnt + 1.0)


def _tc_prep1_body(x_ref, w1_ref, b1_ref, cnt_ref, g1_ref, u1_ref):
  dinv = _dinv_from_cnt(cnt_ref)
  h = jnp.dot(x_ref[...], w1_ref[...], preferred_element_type=jnp.float32)
  g1_ref[...] = jnp.concatenate(
      [dinv * h, jnp.zeros((NP - NN, HH), jnp.float32)], axis=0)
  u1_ref[...] = dinv * dinv * h + b1_ref[...]


def _tc_mid_body(sp_ref, u1_ref, w2_ref, b2_ref, cnt_ref, g2_ref, u2_ref):
  dinv = _dinv_from_cnt(cnt_ref)
  ssum = sp_ref[0, 0:NN, :] + sp_ref[1, 0:NN, :]
  z1 = jnp.maximum(dinv * ssum + u1_ref[...], 0.0)
  h2 = jnp.dot(z1, w2_ref[...], preferred_element_type=jnp.float32)
  g2_ref[...] = jnp.concatenate(
      [dinv * h2, jnp.zeros((NP - NN, HH), jnp.float32)], axis=0)
  u2_ref[...] = dinv * dinv * h2 + b2_ref[...]


def _tc_fin_body(sp_ref, u2_ref, wfc_ref, bfc_ref, cnt_ref, s_ref, t_ref):
  dinv = _dinv_from_cnt(cnt_ref)
  ssum = sp_ref[0, 0:NN, :] + sp_ref[1, 0:NN, :]
  z2 = dinv * ssum + u2_ref[...]
  s_ref[...] = jnp.dot(z2, wfc_ref[0:HH, 0], preferred_element_type=jnp.float32) + bfc_ref[...]
  t_ref[...] = jnp.dot(z2, wfc_ref[HH:2 * HH, 0], preferred_element_type=jnp.float32)


_tc_prep1 = pl.pallas_call(
    _tc_prep1_body,
    out_shape=[jax.ShapeDtypeStruct((NP, HH), jnp.float32),
               jax.ShapeDtypeStruct((NN, HH), jnp.float32)],
)

_tc_mid = pl.pallas_call(
    _tc_mid_body,
    out_shape=[jax.ShapeDtypeStruct((NP, HH), jnp.float32),
               jax.ShapeDtypeStruct((NN, HH), jnp.float32)],
)

_tc_fin = pl.pallas_call(
    _tc_fin_body,
    out_shape=[jax.ShapeDtypeStruct((NN,), jnp.float32),
               jax.ShapeDtypeStruct((NN,), jnp.float32)],
)


def kernel(x, edge_index, W1, b1, W2, b2, Wfc, bfc):
  src = edge_index[0]
  dst = edge_index[1]
  npad = EPAD - EE
  srcp = jnp.concatenate([src, jnp.zeros((npad,), jnp.int32)])
  dstp = jnp.concatenate([dst, jnp.full((npad,), NN, jnp.int32)])
  src3 = srcp.reshape(NWORK, NCH, CHUNK)
  dst3 = dstp.reshape(NWORK, NCH, CHUNK)
  ei3 = edge_index.reshape(2, NWORK, EW)

  zeros16 = jnp.zeros((NP, LL), jnp.float32)
  ones16 = jnp.ones((CHUNK, LL), jnp.float32)
  zeros64 = jnp.zeros((NP, HH), jnp.float32)

  cnt_part = _sc_degree(dst3, zeros16, ones16)
  g1, u1 = _tc_prep1(x, W1, b1, cnt_part)
  s1_part = _sc_msgpass(g1, src3, dst3, zeros64)
  g2, u2 = _tc_mid(s1_part, u1, W2, b2, cnt_part)
  s2_part = _sc_msgpass(g2, src3, dst3, zeros64)
  s_tab, t_tab = _tc_fin(s2_part, u2, Wfc, bfc, cnt_part)
  dec = _sc_decode(s_tab, t_tab, ei3)
  return dec.reshape(EE, 1)


# speedup vs baseline: 1.0452x; 1.0452x over previous
import functools

import jax
import jax.numpy as jnp
from jax import lax
from jax.experimental import pallas as pl
from jax.experimental.pallas import tpu as pltpu
from jax.experimental.pallas import tpu_sc as plsc

NN = 10000
EE = 320000
DD = 128
HH = 64
NC, NS, LL = 2, 16, 16
NWORK = NC * NS
CHUNK = 128
NCH = 80
NBUF = 2
NGRP = NCH // NBUF
EPAD = NWORK * NCH * CHUNK
ROWS_PER_TILE = 632
NP = NS * ROWS_PER_TILE
EW = EE // NWORK

_MESH = plsc.VectorSubcoreMesh(
    core_axis_name="c", subcore_axis_name="s", num_cores=NC, num_subcores=NS)


def _wid():
  return lax.axis_index("c") * NS + lax.axis_index("s")


@functools.partial(
    pl.kernel,
    out_type=jax.ShapeDtypeStruct((NC, NP, LL), jnp.float32),
    mesh=_MESH,
    compiler_params=pltpu.CompilerParams(use_tc_tiling_on_sc=False),
    scratch_types=[
        pltpu.VMEM((NCH, CHUNK), jnp.int32),
        pltpu.VMEM((CHUNK, LL), jnp.float32),
        pltpu.VMEM_SHARED((NP, LL), jnp.float32),
    ],
)
def _sc_degree(dst3, zeros16, ones16, cnt_out, dst_l, ones_v, acc):
  c = lax.axis_index("c")
  s = lax.axis_index("s")
  w = _wid()
  rbase = s * ROWS_PER_TILE
  pltpu.sync_copy(zeros16.at[pl.ds(rbase, ROWS_PER_TILE)],
                  acc.at[pl.ds(rbase, ROWS_PER_TILE)])
  pltpu.sync_copy(ones16, ones_v)
  pltpu.sync_copy(dst3.at[w], dst_l)
  plsc.subcore_barrier()

  @pl.loop(0, NCH)
  def _(j):
    pltpu.sync_copy(ones_v, acc.at[dst_l.at[j]], add=True)

  plsc.subcore_barrier()
  pltpu.sync_copy(acc.at[pl.ds(rbase, ROWS_PER_TILE)],
                  cnt_out.at[c, pl.ds(rbase, ROWS_PER_TILE)])


@functools.partial(
    pl.kernel,
    out_type=jax.ShapeDtypeStruct((NC, NP, HH), jnp.float32),
    mesh=_MESH,
    compiler_params=pltpu.CompilerParams(use_tc_tiling_on_sc=False),
    scratch_types=[
        pltpu.VMEM((NCH, CHUNK), jnp.int32),
        pltpu.VMEM((NCH, CHUNK), jnp.int32),
        [pltpu.VMEM((CHUNK, HH), jnp.float32) for _ in range(NBUF)],
        [pltpu.SemaphoreType.DMA for _ in range(NBUF)],
        [pltpu.SemaphoreType.DMA for _ in range(NBUF)],
        pltpu.VMEM_SHARED((NP, HH), jnp.float32),
        pltpu.VMEM_SHARED((NP, HH), jnp.float32),
    ],
)
def _sc_msgpass(g_tab, src3, dst3, zeros64, s_out,
                src_l, dst_l, rows, gsem, ssem, acc, g_sp):
  c = lax.axis_index("c")
  s = lax.axis_index("s")
  w = _wid()
  rbase = s * ROWS_PER_TILE
  pltpu.sync_copy(g_tab.at[pl.ds(rbase, ROWS_PER_TILE)],
                  g_sp.at[pl.ds(rbase, ROWS_PER_TILE)])
  pltpu.sync_copy(zeros64.at[pl.ds(rbase, ROWS_PER_TILE)],
                  acc.at[pl.ds(rbase, ROWS_PER_TILE)])
  pltpu.sync_copy(src3.at[w], src_l)
  pltpu.sync_copy(dst3.at[w], dst_l)
  plsc.subcore_barrier()

  rows0, rows1 = rows[0], rows[1]
  sem0, sem1 = gsem[0], gsem[1]
  pltpu.async_copy(g_sp.at[src_l.at[0]], rows0, sem0)

  @pl.loop(0, NCH - 1)
  def _(j):
    even = j % 2 == 0

    def do(cur, nxt, sem_cur, sem_nxt):
      pltpu.async_copy(g_sp.at[src_l.at[j + 1]], nxt, sem_nxt)
      pltpu.make_async_copy(g_sp.at[src_l.at[j]], cur, sem_cur).wait()
      pltpu.sync_copy(cur, acc.at[dst_l.at[j]], add=True)

    @pl.when(even)
    def _():
      do(rows0, rows1, sem0, sem1)

    @pl.when(jnp.logical_not(even))
    def _():
      do(rows1, rows0, sem1, sem0)

  last = NCH - 1
  pltpu.make_async_copy(g_sp.at[src_l.at[last]], rows1, sem1).wait()
  pltpu.sync_copy(rows1, acc.at[dst_l.at[last]], add=True)

  plsc.subcore_barrier()
  pltpu.sync_copy(acc.at[pl.ds(rbase, ROWS_PER_TILE)],
                  s_out.at[c, pl.ds(rbase, ROWS_PER_TILE)])


@functools.partial(
    pl.kernel,
    out_type=jax.ShapeDtypeStruct((NWORK, EW), jnp.float32),
    mesh=_MESH,
    compiler_params=pltpu.CompilerParams(
        use_tc_tiling_on_sc=False, needs_layout_passes=False),
    scratch_types=[
        pltpu.VMEM((NN,), jnp.float32),
        pltpu.VMEM((NN,), jnp.float32),
        pltpu.VMEM((EW,), jnp.int32),
        pltpu.VMEM((EW,), jnp.int32),
        pltpu.VMEM((EW,), jnp.float32),
    ],
)
def _sc_decode(s_tab, t_tab, ei3, dec_out, s_l, t_l, src_l, dst_l, ob):
  w = _wid()
  pltpu.sync_copy(s_tab, s_l)
  pltpu.sync_copy(t_tab, t_l)
  pltpu.sync_copy(ei3.at[0, w], src_l)
  pltpu.sync_copy(ei3.at[1, w], dst_l)

  @pl.loop(0, EW // LL, unroll=8)
  def _(i):
    si = src_l[pl.ds(i * LL, LL)]
    di = dst_l[pl.ds(i * LL, LL)]
    sv = plsc.load_gather(s_l, [si])
    tv = plsc.load_gather(t_l, [di])
    y = sv + tv
    ob[pl.ds(i * LL, LL)] = 1.0 / (1.0 + jnp.exp(-y))

  pltpu.sync_copy(ob, dec_out.at[w])


def _dinv_from_cnt(cnt_ref):
  cnt = cnt_ref[0, 0:NN, 0:1] + cnt_ref[1, 0:NN, 0:1]
  return lax.rsqrt(cnt + 1.0)


def _tc_prep1_body(x_ref, w1_ref, b1_ref, cnt_ref, g1_ref, u1_ref):
  dinv = _dinv_from_cnt(cnt_ref)
  h = jnp.dot(x_ref[...], w1_ref[...], preferred_element_type=jnp.float32)
  g1_ref[...] = jnp.concatenate(
      [dinv * h, jnp.zeros((NP - NN, HH), jnp.float32)], axis=0)
  u1_ref[...] = dinv * dinv * h + b1_ref[...]


def _tc_mid_body(sp_ref, u1_ref, w2_ref, b2_ref, cnt_ref, g2_ref, u2_ref):
  dinv = _dinv_from_cnt(cnt_ref)
  ssum = sp_ref[0, 0:NN, :] + sp_ref[1, 0:NN, :]
  z1 = jnp.maximum(dinv * ssum + u1_ref[...], 0.0)
  h2 = jnp.dot(z1, w2_ref[...], preferred_element_type=jnp.float32)
  g2_ref[...] = jnp.concatenate(
      [dinv * h2, jnp.zeros((NP - NN, HH), jnp.float32)], axis=0)
  u2_ref[...] = dinv * dinv * h2 + b2_ref[...]


def _tc_fin_body(sp_ref, u2_ref, wfc_ref, bfc_ref, cnt_ref, s_ref, t_ref):
  dinv = _dinv_from_cnt(cnt_ref)
  ssum = sp_ref[0, 0:NN, :] + sp_ref[1, 0:NN, :]
  z2 = dinv * ssum + u2_ref[...]
  s_ref[...] = jnp.dot(z2, wfc_ref[0:HH, 0], preferred_element_type=jnp.float32) + bfc_ref[...]
  t_ref[...] = jnp.dot(z2, wfc_ref[HH:2 * HH, 0], preferred_element_type=jnp.float32)


_tc_prep1 = pl.pallas_call(
    _tc_prep1_body,
    out_shape=[jax.ShapeDtypeStruct((NP, HH), jnp.float32),
               jax.ShapeDtypeStruct((NN, HH), jnp.float32)],
)

_tc_mid = pl.pallas_call(
    _tc_mid_body,
    out_shape=[jax.ShapeDtypeStruct((NP, HH), jnp.float32),
               jax.ShapeDtypeStruct((NN, HH), jnp.float32)],
)

_tc_fin = pl.pallas_call(
    _tc_fin_body,
    out_shape=[jax.ShapeDtypeStruct((NN,), jnp.float32),
               jax.ShapeDtypeStruct((NN,), jnp.float32)],
)


def kernel(x, edge_index, W1, b1, W2, b2, Wfc, bfc):
  src = edge_index[0]
  dst = edge_index[1]
  npad = EPAD - EE
  srcp = jnp.concatenate([src, jnp.zeros((npad,), jnp.int32)])
  dstp = jnp.concatenate([dst, jnp.full((npad,), NN, jnp.int32)])
  src3 = srcp.reshape(NWORK, NCH, CHUNK)
  dst3 = dstp.reshape(NWORK, NCH, CHUNK)
  ei3 = edge_index.reshape(2, NWORK, EW)

  zeros16 = jnp.zeros((NP, LL), jnp.float32)
  ones16 = jnp.ones((CHUNK, LL), jnp.float32)
  zeros64 = jnp.zeros((NP, HH), jnp.float32)

  cnt_part = _sc_degree(dst3, zeros16, ones16)
  g1, u1 = _tc_prep1(x, W1, b1, cnt_part)
  s1_part = _sc_msgpass(g1, src3, dst3, zeros64)
  g2, u2 = _tc_mid(s1_part, u1, W2, b2, cnt_part)
  s2_part = _sc_msgpass(g2, src3, dst3, zeros64)
  s_tab, t_tab = _tc_fin(s2_part, u2, Wfc, bfc, cnt_part)
  dec = _sc_decode(s_tab, t_tab, ei3)
  return dec.reshape(EE, 1)


# speedup vs baseline: 1.2082x; 1.1560x over previous
import functools

import jax
import jax.numpy as jnp
from jax import lax
from jax.experimental import pallas as pl
from jax.experimental.pallas import tpu as pltpu
from jax.experimental.pallas import tpu_sc as plsc

NN = 10000
EE = 320000
DD = 128
HH = 64
NC, NS, LL = 2, 16, 16
NWORK = NC * NS
CHUNK = 128
NCH = 80
NBUF = 2
NGRP = NCH // NBUF
EPAD = NWORK * NCH * CHUNK
ROWS_PER_TILE = 632
NP = NS * ROWS_PER_TILE
EW = EE // NWORK

_MESH = plsc.VectorSubcoreMesh(
    core_axis_name="c", subcore_axis_name="s", num_cores=NC, num_subcores=NS)


def _wid():
  return lax.axis_index("c") * NS + lax.axis_index("s")


@functools.partial(
    pl.kernel,
    out_type=jax.ShapeDtypeStruct((NC, NP, LL), jnp.float32),
    mesh=_MESH,
    compiler_params=pltpu.CompilerParams(use_tc_tiling_on_sc=False),
    scratch_types=[
        pltpu.VMEM((NCH, CHUNK), jnp.int32),
        pltpu.VMEM((CHUNK, LL), jnp.float32),
        pltpu.VMEM_SHARED((NP, LL), jnp.float32),
    ],
)
def _sc_degree(dst3, zeros16, ones16, cnt_out, dst_l, ones_v, acc):
  c = lax.axis_index("c")
  s = lax.axis_index("s")
  w = _wid()
  rbase = s * ROWS_PER_TILE
  pltpu.sync_copy(zeros16.at[pl.ds(rbase, ROWS_PER_TILE)],
                  acc.at[pl.ds(rbase, ROWS_PER_TILE)])
  pltpu.sync_copy(ones16, ones_v)
  pltpu.sync_copy(dst3.at[w], dst_l)
  plsc.subcore_barrier()

  @pl.loop(0, NCH)
  def _(j):
    pltpu.sync_copy(ones_v, acc.at[dst_l.at[j]], add=True)

  plsc.subcore_barrier()
  pltpu.sync_copy(acc.at[pl.ds(rbase, ROWS_PER_TILE)],
                  cnt_out.at[c, pl.ds(rbase, ROWS_PER_TILE)])


@functools.partial(
    pl.kernel,
    out_type=jax.ShapeDtypeStruct((NC, NP, HH), jnp.float32),
    mesh=_MESH,
    compiler_params=pltpu.CompilerParams(use_tc_tiling_on_sc=False),
    scratch_types=[
        pltpu.VMEM((NCH, CHUNK), jnp.int32),
        pltpu.VMEM((NCH, CHUNK), jnp.int32),
        [pltpu.VMEM((CHUNK, HH), jnp.float32) for _ in range(NBUF)],
        [pltpu.SemaphoreType.DMA for _ in range(NBUF)],
        [pltpu.SemaphoreType.DMA for _ in range(NBUF)],
        pltpu.VMEM_SHARED((NP, HH), jnp.float32),
        pltpu.VMEM_SHARED((NP, HH), jnp.float32),
    ],
)
def _sc_msgpass(g_tab, src3, dst3, zeros64, s_out,
                src_l, dst_l, rows, gsem, ssem, acc, g_sp):
  c = lax.axis_index("c")
  s = lax.axis_index("s")
  w = _wid()
  rbase = s * ROWS_PER_TILE
  pltpu.sync_copy(g_tab.at[pl.ds(rbase, ROWS_PER_TILE)],
                  g_sp.at[pl.ds(rbase, ROWS_PER_TILE)])
  pltpu.sync_copy(zeros64.at[pl.ds(rbase, ROWS_PER_TILE)],
                  acc.at[pl.ds(rbase, ROWS_PER_TILE)])
  pltpu.sync_copy(src3.at[w], src_l)
  pltpu.sync_copy(dst3.at[w], dst_l)
  plsc.subcore_barrier()

  rows0, rows1 = rows[0], rows[1]
  sem0, sem1 = gsem[0], gsem[1]
  pltpu.async_copy(g_sp.at[src_l.at[0]], rows0, sem0)

  @pl.loop(0, NCH - 1)
  def _(j):
    even = j % 2 == 0

    def do(cur, nxt, sem_cur, sem_nxt):
      pltpu.async_copy(g_sp.at[src_l.at[j + 1]], nxt, sem_nxt)
      pltpu.make_async_copy(g_sp.at[src_l.at[j]], cur, sem_cur).wait()
      pltpu.sync_copy(cur, acc.at[dst_l.at[j]], add=True)

    @pl.when(even)
    def _():
      do(rows0, rows1, sem0, sem1)

    @pl.when(jnp.logical_not(even))
    def _():
      do(rows1, rows0, sem1, sem0)

  last = NCH - 1
  pltpu.make_async_copy(g_sp.at[src_l.at[last]], rows1, sem1).wait()
  pltpu.sync_copy(rows1, acc.at[dst_l.at[last]], add=True)

  plsc.subcore_barrier()
  pltpu.sync_copy(acc.at[pl.ds(rbase, ROWS_PER_TILE)],
                  s_out.at[c, pl.ds(rbase, ROWS_PER_TILE)])


@functools.partial(
    pl.kernel,
    out_type=jax.ShapeDtypeStruct((NWORK, EW), jnp.float32),
    mesh=_MESH,
    compiler_params=pltpu.CompilerParams(
        use_tc_tiling_on_sc=False, needs_layout_passes=False),
    scratch_types=[
        pltpu.VMEM((2 * NCH, CHUNK), jnp.int32),
        pltpu.VMEM((2 * NCH, CHUNK), jnp.int32),
        pltpu.VMEM((CHUNK, LL), jnp.float32),
        pltpu.VMEM((CHUNK, LL), jnp.float32),
        pltpu.SemaphoreType.DMA,
        pltpu.SemaphoreType.DMA,
        pltpu.VMEM((640, LL), jnp.float32),
        pltpu.VMEM((640,), jnp.float32),
        pltpu.VMEM((640,), jnp.float32),
        pltpu.VMEM((640,), jnp.float32),
        pltpu.VMEM((640,), jnp.float32),
        pltpu.VMEM((640,), jnp.float32),
        pltpu.VMEM((NP,), jnp.float32),
        pltpu.VMEM((NP,), jnp.float32),
        pltpu.VMEM((EW,), jnp.int32),
        pltpu.VMEM((EW,), jnp.int32),
        pltpu.VMEM((EW,), jnp.float32),
        pltpu.VMEM_SHARED((NP, LL), jnp.float32),
        pltpu.VMEM_SHARED((NP, LL), jnp.float32),
        pltpu.VMEM_SHARED((NP,), jnp.float32),
        pltpu.VMEM_SHARED((NP,), jnp.float32),
    ],
)
def _sc_scalar_decode(pqw, src3, dst3, zeros16, dinv_p, us_p, ut_p, ei3,
                      dec_out,
                      src_l, dst_l, rows0, rows1, sem0, sem1,
                      accv, dinv_l, us_l, ut_l, sv_l, tv_l,
                      s_l, t_l, srcd, dstd, ob,
                      pq_sp, acc2, s_sp, t_sp):
  s = lax.axis_index("s")
  w = _wid()
  rbase = s * ROWS_PER_TILE
  rpt = pl.ds(rbase, ROWS_PER_TILE)
  pltpu.sync_copy(pqw.at[rpt], pq_sp.at[rpt])
  pltpu.sync_copy(zeros16.at[rpt], acc2.at[rpt])
  pltpu.sync_copy(src3.at[s], src_l.at[pl.ds(0, NCH)])
  pltpu.sync_copy(src3.at[s + NS], src_l.at[pl.ds(NCH, NCH)])
  pltpu.sync_copy(dst3.at[s], dst_l.at[pl.ds(0, NCH)])
  pltpu.sync_copy(dst3.at[s + NS], dst_l.at[pl.ds(NCH, NCH)])
  pltpu.sync_copy(dinv_p.at[rpt], dinv_l.at[pl.ds(0, ROWS_PER_TILE)])
  pltpu.sync_copy(us_p.at[rpt], us_l.at[pl.ds(0, ROWS_PER_TILE)])
  pltpu.sync_copy(ut_p.at[rpt], ut_l.at[pl.ds(0, ROWS_PER_TILE)])
  pltpu.sync_copy(ei3.at[0, w], srcd)
  pltpu.sync_copy(ei3.at[1, w], dstd)
  plsc.subcore_barrier()

  pltpu.async_copy(pq_sp.at[src_l.at[0]], rows0, sem0)

  @pl.loop(0, 2 * NCH - 1)
  def _(j):
    even = j % 2 == 0

    def do(cur, nxt, sem_cur, sem_nxt):
      pltpu.async_copy(pq_sp.at[src_l.at[j + 1]], nxt, sem_nxt)
      pltpu.make_async_copy(pq_sp.at[src_l.at[j]], cur, sem_cur).wait()
      pltpu.sync_copy(cur, acc2.at[dst_l.at[j]], add=True)

    @pl.when(even)
    def _():
      do(rows0, rows1, sem0, sem1)

    @pl.when(jnp.logical_not(even))
    def _():
      do(rows1, rows0, sem1, sem0)

  last = 2 * NCH - 1
  pltpu.make_async_copy(pq_sp.at[src_l.at[last]], rows1, sem1).wait()
  pltpu.sync_copy(rows1, acc2.at[dst_l.at[last]], add=True)

  plsc.subcore_barrier()

  pltpu.sync_copy(acc2.at[rpt], accv.at[pl.ds(0, ROWS_PER_TILE)])
  zeros_i = jnp.zeros((LL,), jnp.int32)
  ones_i = jnp.ones((LL,), jnp.int32)

  @pl.loop(0, 640 // LL)
  def _(i):
    r16 = jnp.arange(LL, dtype=jnp.int32) + i * LL
    pv = plsc.load_gather(accv, [r16, zeros_i])
    qv = plsc.load_gather(accv, [r16, ones_i])
    dv = dinv_l[pl.ds(i * LL, LL)]
    sv_l[pl.ds(i * LL, LL)] = dv * pv + us_l[pl.ds(i * LL, LL)]
    tv_l[pl.ds(i * LL, LL)] = dv * qv + ut_l[pl.ds(i * LL, LL)]

  pltpu.sync_copy(sv_l.at[pl.ds(0, ROWS_PER_TILE)], s_sp.at[rpt])
  pltpu.sync_copy(tv_l.at[pl.ds(0, ROWS_PER_TILE)], t_sp.at[rpt])
  plsc.subcore_barrier()

  pltpu.sync_copy(s_sp, s_l)
  pltpu.sync_copy(t_sp, t_l)

  @pl.loop(0, EW // LL, unroll=8)
  def _(i):
    si = srcd[pl.ds(i * LL, LL)]
    di = dstd[pl.ds(i * LL, LL)]
    sv = plsc.load_gather(s_l, [si])
    tv = plsc.load_gather(t_l, [di])
    y = sv + tv
    ob[pl.ds(i * LL, LL)] = 1.0 / (1.0 + jnp.exp(-y))

  pltpu.sync_copy(ob, dec_out.at[w])


def _dinv_from_cnt(cnt_ref):
  cnt = cnt_ref[0, 0:NN, 0:1] + cnt_ref[1, 0:NN, 0:1]
  return lax.rsqrt(cnt + 1.0)


def _tc_prep1_body(x_ref, w1_ref, b1_ref, cnt_ref, g1_ref, u1_ref):
  dinv = _dinv_from_cnt(cnt_ref)
  h = jnp.dot(x_ref[...], w1_ref[...], preferred_element_type=jnp.float32)
  g1_ref[...] = jnp.concatenate(
      [dinv * h, jnp.zeros((NP - NN, HH), jnp.float32)], axis=0)
  u1_ref[...] = dinv * dinv * h + b1_ref[...]


def _tc_mid_body(sp_ref, u1_ref, w2_ref, b2_ref, wfc_ref, bfc_ref, cnt_ref,
                 pqw_ref, dinv_ref, us_ref, ut_ref):
  dinv = _dinv_from_cnt(cnt_ref)
  ssum = sp_ref[0, 0:NN, :] + sp_ref[1, 0:NN, :]
  z1 = jnp.maximum(dinv * ssum + u1_ref[...], 0.0)
  h2 = jnp.dot(z1, w2_ref[...], preferred_element_type=jnp.float32)
  u2 = dinv * dinv * h2 + b2_ref[...]
  wa = wfc_ref[0:HH, 0]
  wb = wfc_ref[HH:2 * HH, 0]
  p = dinv[:, 0] * jnp.dot(h2, wa, preferred_element_type=jnp.float32)
  q = dinv[:, 0] * jnp.dot(h2, wb, preferred_element_type=jnp.float32)
  pqw = jnp.concatenate(
      [p[:, None], q[:, None], jnp.zeros((NN, LL - 2), jnp.float32)], axis=1)
  pqw_ref[...] = jnp.concatenate(
      [pqw, jnp.zeros((NP - NN, LL), jnp.float32)], axis=0)
  ztail = jnp.zeros((NP - NN,), jnp.float32)
  dinv_ref[...] = jnp.concatenate([dinv[:, 0], ztail])
  us_ref[...] = jnp.concatenate(
      [jnp.dot(u2, wa, preferred_element_type=jnp.float32) + bfc_ref[...],
       ztail])
  ut_ref[...] = jnp.concatenate(
      [jnp.dot(u2, wb, preferred_element_type=jnp.float32), ztail])


_tc_prep1 = pl.pallas_call(
    _tc_prep1_body,
    out_shape=[jax.ShapeDtypeStruct((NP, HH), jnp.float32),
               jax.ShapeDtypeStruct((NN, HH), jnp.float32)],
)

_tc_mid = pl.pallas_call(
    _tc_mid_body,
    out_shape=[jax.ShapeDtypeStruct((NP, LL), jnp.float32),
               jax.ShapeDtypeStruct((NP,), jnp.float32),
               jax.ShapeDtypeStruct((NP,), jnp.float32),
               jax.ShapeDtypeStruct((NP,), jnp.float32)],
)


def kernel(x, edge_index, W1, b1, W2, b2, Wfc, bfc):
  src = edge_index[0]
  dst = edge_index[1]
  npad = EPAD - EE
  srcp = jnp.concatenate([src, jnp.zeros((npad,), jnp.int32)])
  dstp = jnp.concatenate([dst, jnp.full((npad,), NN, jnp.int32)])
  src3 = srcp.reshape(NWORK, NCH, CHUNK)
  dst3 = dstp.reshape(NWORK, NCH, CHUNK)
  ei3 = edge_index.reshape(2, NWORK, EW)

  zeros16 = jnp.zeros((NP, LL), jnp.float32)
  ones16 = jnp.ones((CHUNK, LL), jnp.float32)
  zeros64 = jnp.zeros((NP, HH), jnp.float32)

  cnt_part = _sc_degree(dst3, zeros16, ones16)
  g1, u1 = _tc_prep1(x, W1, b1, cnt_part)
  s1_part = _sc_msgpass(g1, src3, dst3, zeros64)
  pqw, dinv_p, us_p, ut_p = _tc_mid(s1_part, u1, W2, b2, Wfc, bfc, cnt_part)
  dec = _sc_scalar_decode(pqw, src3, dst3, zeros16, dinv_p, us_p, ut_p, ei3)
  return dec.reshape(EE, 1)


# speedup vs baseline: 1.2652x; 1.0472x over previous
import functools

import jax
import jax.numpy as jnp
from jax import lax
from jax.experimental import pallas as pl
from jax.experimental.pallas import tpu as pltpu
from jax.experimental.pallas import tpu_sc as plsc

NN = 10000
EE = 320000
DD = 128
HH = 64
NC, NS, LL = 2, 16, 16
NWORK = NC * NS
CHUNK = 128
NCH = 80
NBUF = 2
NGRP = NCH // NBUF
EPAD = NWORK * NCH * CHUNK
ROWS_PER_TILE = 632
NP = NS * ROWS_PER_TILE
EW = EE // NWORK

_MESH = plsc.VectorSubcoreMesh(
    core_axis_name="c", subcore_axis_name="s", num_cores=NC, num_subcores=NS)


def _wid():
  return lax.axis_index("c") * NS + lax.axis_index("s")


@functools.partial(
    pl.kernel,
    out_type=jax.ShapeDtypeStruct((NC, NP, 2), jnp.float32),
    mesh=_MESH,
    compiler_params=pltpu.CompilerParams(use_tc_tiling_on_sc=False),
    scratch_types=[
        pltpu.VMEM((NCH, CHUNK), jnp.int32),
        pltpu.VMEM((CHUNK, 2), jnp.float32),
        pltpu.VMEM_SHARED((NP, 2), jnp.float32),
    ],
)
def _sc_degree(dst3, zeros16, ones16, cnt_out, dst_l, ones_v, acc):
  c = lax.axis_index("c")
  s = lax.axis_index("s")
  w = _wid()
  rbase = s * ROWS_PER_TILE
  pltpu.sync_copy(zeros16.at[pl.ds(rbase, ROWS_PER_TILE)],
                  acc.at[pl.ds(rbase, ROWS_PER_TILE)])
  pltpu.sync_copy(ones16, ones_v)
  pltpu.sync_copy(dst3.at[w], dst_l)
  plsc.subcore_barrier()

  @pl.loop(0, NCH)
  def _(j):
    pltpu.sync_copy(ones_v, acc.at[dst_l.at[j]], add=True)

  plsc.subcore_barrier()
  pltpu.sync_copy(acc.at[pl.ds(rbase, ROWS_PER_TILE)],
                  cnt_out.at[c, pl.ds(rbase, ROWS_PER_TILE)])


@functools.partial(
    pl.kernel,
    out_type=jax.ShapeDtypeStruct((NC, NP, HH), jnp.float32),
    mesh=_MESH,
    compiler_params=pltpu.CompilerParams(use_tc_tiling_on_sc=False),
    scratch_types=[
        pltpu.VMEM((NCH, CHUNK), jnp.int32),
        pltpu.VMEM((NCH, CHUNK), jnp.int32),
        [pltpu.VMEM((CHUNK, HH), jnp.float32) for _ in range(NBUF)],
        [pltpu.SemaphoreType.DMA for _ in range(NBUF)],
        [pltpu.SemaphoreType.DMA for _ in range(NBUF)],
        pltpu.VMEM_SHARED((NP, HH), jnp.float32),
        pltpu.VMEM_SHARED((NP, HH), jnp.float32),
    ],
)
def _sc_msgpass(g_tab, src3, dst3, zeros64, s_out,
                src_l, dst_l, rows, gsem, ssem, acc, g_sp):
  c = lax.axis_index("c")
  s = lax.axis_index("s")
  w = _wid()
  rbase = s * ROWS_PER_TILE
  pltpu.sync_copy(g_tab.at[pl.ds(rbase, ROWS_PER_TILE)],
                  g_sp.at[pl.ds(rbase, ROWS_PER_TILE)])
  pltpu.sync_copy(zeros64.at[pl.ds(rbase, ROWS_PER_TILE)],
                  acc.at[pl.ds(rbase, ROWS_PER_TILE)])
  pltpu.sync_copy(src3.at[w], src_l)
  pltpu.sync_copy(dst3.at[w], dst_l)
  plsc.subcore_barrier()

  rows0, rows1 = rows[0], rows[1]
  sem0, sem1 = gsem[0], gsem[1]
  pltpu.async_copy(g_sp.at[src_l.at[0]], rows0, sem0)

  @pl.loop(0, NCH - 1)
  def _(j):
    even = j % 2 == 0

    def do(cur, nxt, sem_cur, sem_nxt):
      pltpu.async_copy(g_sp.at[src_l.at[j + 1]], nxt, sem_nxt)
      pltpu.make_async_copy(g_sp.at[src_l.at[j]], cur, sem_cur).wait()
      pltpu.sync_copy(cur, acc.at[dst_l.at[j]], add=True)

    @pl.when(even)
    def _():
      do(rows0, rows1, sem0, sem1)

    @pl.when(jnp.logical_not(even))
    def _():
      do(rows1, rows0, sem1, sem0)

  last = NCH - 1
  pltpu.make_async_copy(g_sp.at[src_l.at[last]], rows1, sem1).wait()
  pltpu.sync_copy(rows1, acc.at[dst_l.at[last]], add=True)

  plsc.subcore_barrier()
  pltpu.sync_copy(acc.at[pl.ds(rbase, ROWS_PER_TILE)],
                  s_out.at[c, pl.ds(rbase, ROWS_PER_TILE)])


@functools.partial(
    pl.kernel,
    out_type=jax.ShapeDtypeStruct((NWORK, EW), jnp.float32),
    mesh=_MESH,
    compiler_params=pltpu.CompilerParams(
        use_tc_tiling_on_sc=False, needs_layout_passes=False),
    scratch_types=[
        pltpu.VMEM((2 * NCH, CHUNK), jnp.int32),
        pltpu.VMEM((2 * NCH, CHUNK), jnp.int32),
        pltpu.VMEM((CHUNK, 2), jnp.float32),
        pltpu.VMEM((CHUNK, 2), jnp.float32),
        pltpu.SemaphoreType.DMA,
        pltpu.SemaphoreType.DMA,
        pltpu.VMEM((640, 2), jnp.float32),
        pltpu.VMEM((640,), jnp.float32),
        pltpu.VMEM((640,), jnp.float32),
        pltpu.VMEM((640,), jnp.float32),
        pltpu.VMEM((640,), jnp.float32),
        pltpu.VMEM((640,), jnp.float32),
        pltpu.VMEM((NP,), jnp.float32),
        pltpu.VMEM((NP,), jnp.float32),
        pltpu.VMEM((EW,), jnp.int32),
        pltpu.VMEM((EW,), jnp.int32),
        pltpu.VMEM((EW,), jnp.float32),
        pltpu.VMEM_SHARED((NP, 2), jnp.float32),
        pltpu.VMEM_SHARED((NP, 2), jnp.float32),
        pltpu.VMEM_SHARED((NP,), jnp.float32),
        pltpu.VMEM_SHARED((NP,), jnp.float32),
    ],
)
def _sc_scalar_decode(pqw, src3, dst3, zeros2, dinv_p, us_p, ut_p, ei3,
                      dec_out,
                      src_l, dst_l, rows0, rows1, sem0, sem1,
                      accv, dinv_l, us_l, ut_l, sv_l, tv_l,
                      s_l, t_l, srcd, dstd, ob,
                      pq_sp, acc2, s_sp, t_sp):
  s = lax.axis_index("s")
  w = _wid()
  rbase = s * ROWS_PER_TILE
  rpt = pl.ds(rbase, ROWS_PER_TILE)
  pltpu.sync_copy(pqw.at[rpt], pq_sp.at[rpt])
  pltpu.sync_copy(zeros2.at[rpt], acc2.at[rpt])
  pltpu.sync_copy(src3.at[s], src_l.at[pl.ds(0, NCH)])
  pltpu.sync_copy(src3.at[s + NS], src_l.at[pl.ds(NCH, NCH)])
  pltpu.sync_copy(dst3.at[s], dst_l.at[pl.ds(0, NCH)])
  pltpu.sync_copy(dst3.at[s + NS], dst_l.at[pl.ds(NCH, NCH)])
  pltpu.sync_copy(dinv_p.at[rpt], dinv_l.at[pl.ds(0, ROWS_PER_TILE)])
  pltpu.sync_copy(us_p.at[rpt], us_l.at[pl.ds(0, ROWS_PER_TILE)])
  pltpu.sync_copy(ut_p.at[rpt], ut_l.at[pl.ds(0, ROWS_PER_TILE)])
  pltpu.sync_copy(ei3.at[0, w], srcd)
  pltpu.sync_copy(ei3.at[1, w], dstd)
  plsc.subcore_barrier()

  pltpu.async_copy(pq_sp.at[src_l.at[0]], rows0, sem0)

  @pl.loop(0, 2 * NCH - 1)
  def _(j):
    even = j % 2 == 0

    def do(cur, nxt, sem_cur, sem_nxt):
      pltpu.async_copy(pq_sp.at[src_l.at[j + 1]], nxt, sem_nxt)
      pltpu.make_async_copy(pq_sp.at[src_l.at[j]], cur, sem_cur).wait()
      pltpu.sync_copy(cur, acc2.at[dst_l.at[j]], add=True)

    @pl.when(even)
    def _():
      do(rows0, rows1, sem0, sem1)

    @pl.when(jnp.logical_not(even))
    def _():
      do(rows1, rows0, sem1, sem0)

  last = 2 * NCH - 1
  pltpu.make_async_copy(pq_sp.at[src_l.at[last]], rows1, sem1).wait()
  pltpu.sync_copy(rows1, acc2.at[dst_l.at[last]], add=True)

  plsc.subcore_barrier()

  pltpu.sync_copy(acc2.at[rpt], accv.at[pl.ds(0, ROWS_PER_TILE)])
  zeros_i = jnp.zeros((LL,), jnp.int32)
  ones_i = jnp.ones((LL,), jnp.int32)

  @pl.loop(0, 640 // LL)
  def _(i):
    r16 = jnp.arange(LL, dtype=jnp.int32) + i * LL
    pv = plsc.load_gather(accv, [r16, zeros_i])
    qv = plsc.load_gather(accv, [r16, ones_i])
    dv = dinv_l[pl.ds(i * LL, LL)]
    sv_l[pl.ds(i * LL, LL)] = dv * pv + us_l[pl.ds(i * LL, LL)]
    tv_l[pl.ds(i * LL, LL)] = dv * qv + ut_l[pl.ds(i * LL, LL)]

  pltpu.sync_copy(sv_l.at[pl.ds(0, ROWS_PER_TILE)], s_sp.at[rpt])
  pltpu.sync_copy(tv_l.at[pl.ds(0, ROWS_PER_TILE)], t_sp.at[rpt])
  plsc.subcore_barrier()

  pltpu.sync_copy(s_sp, s_l)
  pltpu.sync_copy(t_sp, t_l)

  @pl.loop(0, EW // LL, unroll=8)
  def _(i):
    si = srcd[pl.ds(i * LL, LL)]
    di = dstd[pl.ds(i * LL, LL)]
    sv = plsc.load_gather(s_l, [si])
    tv = plsc.load_gather(t_l, [di])
    y = sv + tv
    ob[pl.ds(i * LL, LL)] = 1.0 / (1.0 + jnp.exp(-y))

  pltpu.sync_copy(ob, dec_out.at[w])


def _dinv_from_cnt(cnt_ref):
  cnt = cnt_ref[0, 0:NN, 0:1] + cnt_ref[1, 0:NN, 0:1]
  return lax.rsqrt(cnt + 1.0)


def _tc_prep1_body(x_ref, w1_ref, b1_ref, cnt_ref, g1_ref, u1_ref):
  dinv = _dinv_from_cnt(cnt_ref)
  h = jnp.dot(x_ref[...], w1_ref[...], preferred_element_type=jnp.float32)
  g1_ref[...] = jnp.concatenate(
      [dinv * h, jnp.zeros((NP - NN, HH), jnp.float32)], axis=0)
  u1_ref[...] = dinv * dinv * h + b1_ref[...]


def _tc_mid_body(sp_ref, u1_ref, w2_ref, b2_ref, wfc_ref, bfc_ref, cnt_ref,
                 pqw_ref, dinv_ref, us_ref, ut_ref):
  dinv = _dinv_from_cnt(cnt_ref)
  ssum = sp_ref[0, 0:NN, :] + sp_ref[1, 0:NN, :]
  z1 = jnp.maximum(dinv * ssum + u1_ref[...], 0.0)
  h2 = jnp.dot(z1, w2_ref[...], preferred_element_type=jnp.float32)
  u2 = dinv * dinv * h2 + b2_ref[...]
  wa = wfc_ref[0:HH, 0]
  wb = wfc_ref[HH:2 * HH, 0]
  p = dinv[:, 0] * jnp.dot(h2, wa, preferred_element_type=jnp.float32)
  q = dinv[:, 0] * jnp.dot(h2, wb, preferred_element_type=jnp.float32)
  pqw = jnp.concatenate([p[:, None], q[:, None]], axis=1)
  pqw_ref[...] = jnp.concatenate(
      [pqw, jnp.zeros((NP - NN, 2), jnp.float32)], axis=0)
  ztail = jnp.zeros((NP - NN,), jnp.float32)
  dinv_ref[...] = jnp.concatenate([dinv[:, 0], ztail])
  us_ref[...] = jnp.concatenate(
      [jnp.dot(u2, wa, preferred_element_type=jnp.float32) + bfc_ref[...],
       ztail])
  ut_ref[...] = jnp.concatenate(
      [jnp.dot(u2, wb, preferred_element_type=jnp.float32), ztail])


_tc_prep1 = pl.pallas_call(
    _tc_prep1_body,
    out_shape=[jax.ShapeDtypeStruct((NP, HH), jnp.float32),
               jax.ShapeDtypeStruct((NN, HH), jnp.float32)],
)

_tc_mid = pl.pallas_call(
    _tc_mid_body,
    out_shape=[jax.ShapeDtypeStruct((NP, 2), jnp.float32),
               jax.ShapeDtypeStruct((NP,), jnp.float32),
               jax.ShapeDtypeStruct((NP,), jnp.float32),
               jax.ShapeDtypeStruct((NP,), jnp.float32)],
)


def kernel(x, edge_index, W1, b1, W2, b2, Wfc, bfc):
  src = edge_index[0]
  dst = edge_index[1]
  npad = EPAD - EE
  srcp = jnp.concatenate([src, jnp.zeros((npad,), jnp.int32)])
  dstp = jnp.concatenate([dst, jnp.full((npad,), NN, jnp.int32)])
  src3 = srcp.reshape(NWORK, NCH, CHUNK)
  dst3 = dstp.reshape(NWORK, NCH, CHUNK)
  ei3 = edge_index.reshape(2, NWORK, EW)

  zeros2 = jnp.zeros((NP, 2), jnp.float32)
  ones2 = jnp.ones((CHUNK, 2), jnp.float32)
  zeros64 = jnp.zeros((NP, HH), jnp.float32)

  cnt_part = _sc_degree(dst3, zeros2, ones2)
  g1, u1 = _tc_prep1(x, W1, b1, cnt_part)
  s1_part = _sc_msgpass(g1, src3, dst3, zeros64)
  pqw, dinv_p, us_p, ut_p = _tc_mid(s1_part, u1, W2, b2, Wfc, bfc, cnt_part)
  dec = _sc_scalar_decode(pqw, src3, dst3, zeros2, dinv_p, us_p, ut_p, ei3)
  return dec.reshape(EE, 1)


# speedup vs baseline: 1.2669x; 1.0013x over previous
import functools

import jax
import jax.numpy as jnp
from jax import lax
from jax.experimental import pallas as pl
from jax.experimental.pallas import tpu as pltpu
from jax.experimental.pallas import tpu_sc as plsc

NN = 10000
EE = 320000
DD = 128
HH = 64
NC, NS, LL = 2, 16, 16
NWORK = NC * NS
CHUNK = 128
NCH = 80
NBUF = 2
NGRP = NCH // NBUF
EPAD = NWORK * NCH * CHUNK
ROWS_PER_TILE = 632
NP = NS * ROWS_PER_TILE
EW = EE // NWORK

_MESH = plsc.VectorSubcoreMesh(
    core_axis_name="c", subcore_axis_name="s", num_cores=NC, num_subcores=NS)


def _wid():
  return lax.axis_index("c") * NS + lax.axis_index("s")


@functools.partial(
    pl.kernel,
    out_type=jax.ShapeDtypeStruct((NC, NP), jnp.float32),
    mesh=_MESH,
    compiler_params=pltpu.CompilerParams(
        use_tc_tiling_on_sc=False, needs_layout_passes=False),
    scratch_types=[
        pltpu.VMEM((NCH, CHUNK), jnp.int32),
        pltpu.VMEM((NP,), jnp.float32),
        pltpu.VMEM((ROWS_PER_TILE,), jnp.float32),
        pltpu.VMEM((ROWS_PER_TILE,), jnp.float32),
        pltpu.VMEM_SHARED((NS, NP), jnp.float32),
    ],
)
def _sc_degree(dst3, cnt_out, dst_l, cnt_l, accb, tbuf, stage):
  c = lax.axis_index("c")
  s = lax.axis_index("s")
  w = _wid()
  rbase = s * ROWS_PER_TILE
  rpt = pl.ds(rbase, ROWS_PER_TILE)
  pltpu.sync_copy(dst3.at[w], dst_l)

  @pl.loop(0, NP // LL)
  def _(i):
    cnt_l[pl.ds(i * LL, LL)] = jnp.zeros((LL,), jnp.float32)

  ones = jnp.ones((LL,), jnp.float32)

  @pl.loop(0, NCH)
  def _(j):
    for k in range(CHUNK // LL):
      dv = dst_l[j, pl.ds(k * LL, LL)]
      plsc.addupdate_scatter(cnt_l, [dv], ones)

  pltpu.sync_copy(cnt_l, stage.at[s])
  plsc.subcore_barrier()

  @pl.loop(0, ROWS_PER_TILE // LL)
  def _(i):
    accb[pl.ds(i * LL, LL)] = jnp.zeros((LL,), jnp.float32)

  for t in range(NS):
    pltpu.sync_copy(stage.at[t, rpt], tbuf)

    @pl.loop(0, ROWS_PER_TILE // LL)
    def _(i):
      sl = pl.ds(i * LL, LL)
      accb[sl] = accb[sl] + tbuf[sl]

  pltpu.sync_copy(accb, cnt_out.at[c, rpt])


@functools.partial(
    pl.kernel,
    out_type=jax.ShapeDtypeStruct((NC, NP, HH), jnp.float32),
    mesh=_MESH,
    compiler_params=pltpu.CompilerParams(use_tc_tiling_on_sc=False),
    scratch_types=[
        pltpu.VMEM((NCH, CHUNK), jnp.int32),
        pltpu.VMEM((NCH, CHUNK), jnp.int32),
        [pltpu.VMEM((CHUNK, HH), jnp.float32) for _ in range(NBUF)],
        [pltpu.SemaphoreType.DMA for _ in range(NBUF)],
        [pltpu.SemaphoreType.DMA for _ in range(NBUF)],
        pltpu.VMEM_SHARED((NP, HH), jnp.float32),
        pltpu.VMEM_SHARED((NP, HH), jnp.float32),
    ],
)
def _sc_msgpass(g_tab, src3, dst3, zeros64, s_out,
                src_l, dst_l, rows, gsem, ssem, acc, g_sp):
  c = lax.axis_index("c")
  s = lax.axis_index("s")
  w = _wid()
  rbase = s * ROWS_PER_TILE
  pltpu.sync_copy(g_tab.at[pl.ds(rbase, ROWS_PER_TILE)],
                  g_sp.at[pl.ds(rbase, ROWS_PER_TILE)])
  pltpu.sync_copy(zeros64.at[pl.ds(rbase, ROWS_PER_TILE)],
                  acc.at[pl.ds(rbase, ROWS_PER_TILE)])
  pltpu.sync_copy(src3.at[w], src_l)
  pltpu.sync_copy(dst3.at[w], dst_l)
  plsc.subcore_barrier()

  rows0, rows1 = rows[0], rows[1]
  sem0, sem1 = gsem[0], gsem[1]
  pltpu.async_copy(g_sp.at[src_l.at[0]], rows0, sem0)

  @pl.loop(0, NCH - 1)
  def _(j):
    even = j % 2 == 0

    def do(cur, nxt, sem_cur, sem_nxt):
      pltpu.async_copy(g_sp.at[src_l.at[j + 1]], nxt, sem_nxt)
      pltpu.make_async_copy(g_sp.at[src_l.at[j]], cur, sem_cur).wait()
      pltpu.sync_copy(cur, acc.at[dst_l.at[j]], add=True)

    @pl.when(even)
    def _():
      do(rows0, rows1, sem0, sem1)

    @pl.when(jnp.logical_not(even))
    def _():
      do(rows1, rows0, sem1, sem0)

  last = NCH - 1
  pltpu.make_async_copy(g_sp.at[src_l.at[last]], rows1, sem1).wait()
  pltpu.sync_copy(rows1, acc.at[dst_l.at[last]], add=True)

  plsc.subcore_barrier()
  pltpu.sync_copy(acc.at[pl.ds(rbase, ROWS_PER_TILE)],
                  s_out.at[c, pl.ds(rbase, ROWS_PER_TILE)])


@functools.partial(
    pl.kernel,
    out_type=jax.ShapeDtypeStruct((NWORK, EW), jnp.float32),
    mesh=_MESH,
    compiler_params=pltpu.CompilerParams(
        use_tc_tiling_on_sc=False, needs_layout_passes=False),
    scratch_types=[
        pltpu.VMEM((2 * NCH, CHUNK), jnp.int32),
        pltpu.VMEM((2 * NCH, CHUNK), jnp.int32),
        pltpu.VMEM((CHUNK, LL), jnp.float32),
        pltpu.VMEM((CHUNK, LL), jnp.float32),
        pltpu.SemaphoreType.DMA,
        pltpu.SemaphoreType.DMA,
        pltpu.VMEM((640, LL), jnp.float32),
        pltpu.VMEM((640,), jnp.float32),
        pltpu.VMEM((640,), jnp.float32),
        pltpu.VMEM((640,), jnp.float32),
        pltpu.VMEM((640,), jnp.float32),
        pltpu.VMEM((640,), jnp.float32),
        pltpu.VMEM((NP,), jnp.float32),
        pltpu.VMEM((NP,), jnp.float32),
        pltpu.VMEM((EW,), jnp.int32),
        pltpu.VMEM((EW,), jnp.int32),
        pltpu.VMEM((EW,), jnp.float32),
        pltpu.VMEM_SHARED((NP, LL), jnp.float32),
        pltpu.VMEM_SHARED((NP, LL), jnp.float32),
        pltpu.VMEM_SHARED((NP,), jnp.float32),
        pltpu.VMEM_SHARED((NP,), jnp.float32),
    ],
)
def _sc_scalar_decode(pqw, src3, dst3, zeros2, dinv_p, us_p, ut_p, ei3,
                      dec_out,
                      src_l, dst_l, rows0, rows1, sem0, sem1,
                      accv, dinv_l, us_l, ut_l, sv_l, tv_l,
                      s_l, t_l, srcd, dstd, ob,
                      pq_sp, acc2, s_sp, t_sp):
  s = lax.axis_index("s")
  w = _wid()
  rbase = s * ROWS_PER_TILE
  rpt = pl.ds(rbase, ROWS_PER_TILE)
  pltpu.sync_copy(pqw.at[rpt], pq_sp.at[rpt])
  pltpu.sync_copy(zeros2.at[rpt], acc2.at[rpt])
  pltpu.sync_copy(src3.at[s], src_l.at[pl.ds(0, NCH)])
  pltpu.sync_copy(src3.at[s + NS], src_l.at[pl.ds(NCH, NCH)])
  pltpu.sync_copy(dst3.at[s], dst_l.at[pl.ds(0, NCH)])
  pltpu.sync_copy(dst3.at[s + NS], dst_l.at[pl.ds(NCH, NCH)])
  pltpu.sync_copy(dinv_p.at[rpt], dinv_l.at[pl.ds(0, ROWS_PER_TILE)])
  pltpu.sync_copy(us_p.at[rpt], us_l.at[pl.ds(0, ROWS_PER_TILE)])
  pltpu.sync_copy(ut_p.at[rpt], ut_l.at[pl.ds(0, ROWS_PER_TILE)])
  pltpu.sync_copy(ei3.at[0, w], srcd)
  pltpu.sync_copy(ei3.at[1, w], dstd)
  plsc.subcore_barrier()

  pltpu.async_copy(pq_sp.at[src_l.at[0]], rows0, sem0)

  @pl.loop(0, 2 * NCH - 1)
  def _(j):
    even = j % 2 == 0

    def do(cur, nxt, sem_cur, sem_nxt):
      pltpu.async_copy(pq_sp.at[src_l.at[j + 1]], nxt, sem_nxt)
      pltpu.make_async_copy(pq_sp.at[src_l.at[j]], cur, sem_cur).wait()
      pltpu.sync_copy(cur, acc2.at[dst_l.at[j]], add=True)

    @pl.when(even)
    def _():
      do(rows0, rows1, sem0, sem1)

    @pl.when(jnp.logical_not(even))
    def _():
      do(rows1, rows0, sem1, sem0)

  last = 2 * NCH - 1
  pltpu.make_async_copy(pq_sp.at[src_l.at[last]], rows1, sem1).wait()
  pltpu.sync_copy(rows1, acc2.at[dst_l.at[last]], add=True)

  plsc.subcore_barrier()

  pltpu.sync_copy(acc2.at[rpt], accv.at[pl.ds(0, ROWS_PER_TILE)])
  zeros_i = jnp.zeros((LL,), jnp.int32)
  ones_i = jnp.ones((LL,), jnp.int32)

  @pl.loop(0, 640 // LL)
  def _(i):
    r16 = jnp.arange(LL, dtype=jnp.int32) + i * LL
    pv = plsc.load_gather(accv, [r16, zeros_i])
    qv = plsc.load_gather(accv, [r16, ones_i])
    dv = dinv_l[pl.ds(i * LL, LL)]
    sv_l[pl.ds(i * LL, LL)] = dv * pv + us_l[pl.ds(i * LL, LL)]
    tv_l[pl.ds(i * LL, LL)] = dv * qv + ut_l[pl.ds(i * LL, LL)]

  pltpu.sync_copy(sv_l.at[pl.ds(0, ROWS_PER_TILE)], s_sp.at[rpt])
  pltpu.sync_copy(tv_l.at[pl.ds(0, ROWS_PER_TILE)], t_sp.at[rpt])
  plsc.subcore_barrier()

  pltpu.sync_copy(s_sp, s_l)
  pltpu.sync_copy(t_sp, t_l)

  @pl.loop(0, EW // LL, unroll=8)
  def _(i):
    si = srcd[pl.ds(i * LL, LL)]
    di = dstd[pl.ds(i * LL, LL)]
    sv = plsc.load_gather(s_l, [si])
    tv = plsc.load_gather(t_l, [di])
    y = sv + tv
    ob[pl.ds(i * LL, LL)] = 1.0 / (1.0 + jnp.exp(-y))

  pltpu.sync_copy(ob, dec_out.at[w])


def _dinv_from_cnt(cnt_ref):
  cnt = cnt_ref[0, 0:NN] + cnt_ref[1, 0:NN]
  return lax.rsqrt(cnt + 1.0)[:, None]


def _tc_prep1_body(x_ref, w1_ref, b1_ref, cnt_ref, g1_ref, u1_ref):
  dinv = _dinv_from_cnt(cnt_ref)
  h = jnp.dot(x_ref[...], w1_ref[...], preferred_element_type=jnp.float32)
  g1_ref[...] = jnp.concatenate(
      [dinv * h, jnp.zeros((NP - NN, HH), jnp.float32)], axis=0)
  u1_ref[...] = dinv * dinv * h + b1_ref[...]


def _tc_mid_body(sp_ref, u1_ref, w2_ref, b2_ref, wfc_ref, bfc_ref, cnt_ref,
                 pqw_ref, dinv_ref, us_ref, ut_ref):
  dinv = _dinv_from_cnt(cnt_ref)
  ssum = sp_ref[0, 0:NN, :] + sp_ref[1, 0:NN, :]
  z1 = jnp.maximum(dinv * ssum + u1_ref[...], 0.0)
  h2 = jnp.dot(z1, w2_ref[...], preferred_element_type=jnp.float32)
  u2 = dinv * dinv * h2 + b2_ref[...]
  wa = wfc_ref[0:HH, 0]
  wb = wfc_ref[HH:2 * HH, 0]
  p = dinv[:, 0] * jnp.dot(h2, wa, preferred_element_type=jnp.float32)
  q = dinv[:, 0] * jnp.dot(h2, wb, preferred_element_type=jnp.float32)
  pqw = jnp.concatenate(
      [p[:, None], q[:, None], jnp.zeros((NN, LL - 2), jnp.float32)], axis=1)
  pqw_ref[...] = jnp.concatenate(
      [pqw, jnp.zeros((NP - NN, LL), jnp.float32)], axis=0)
  ztail = jnp.zeros((NP - NN,), jnp.float32)
  dinv_ref[...] = jnp.concatenate([dinv[:, 0], ztail])
  us_ref[...] = jnp.concatenate(
      [jnp.dot(u2, wa, preferred_element_type=jnp.float32) + bfc_ref[...],
       ztail])
  ut_ref[...] = jnp.concatenate(
      [jnp.dot(u2, wb, preferred_element_type=jnp.float32), ztail])


_tc_prep1 = pl.pallas_call(
    _tc_prep1_body,
    out_shape=[jax.ShapeDtypeStruct((NP, HH), jnp.float32),
               jax.ShapeDtypeStruct((NN, HH), jnp.float32)],
)

_tc_mid = pl.pallas_call(
    _tc_mid_body,
    out_shape=[jax.ShapeDtypeStruct((NP, LL), jnp.float32),
               jax.ShapeDtypeStruct((NP,), jnp.float32),
               jax.ShapeDtypeStruct((NP,), jnp.float32),
               jax.ShapeDtypeStruct((NP,), jnp.float32)],
)


def kernel(x, edge_index, W1, b1, W2, b2, Wfc, bfc):
  src = edge_index[0]
  dst = edge_index[1]
  npad = EPAD - EE
  srcp = jnp.concatenate([src, jnp.zeros((npad,), jnp.int32)])
  dstp = jnp.concatenate([dst, jnp.full((npad,), NN, jnp.int32)])
  src3 = srcp.reshape(NWORK, NCH, CHUNK)
  dst3 = dstp.reshape(NWORK, NCH, CHUNK)
  ei3 = edge_index.reshape(2, NWORK, EW)

  zeros2 = jnp.zeros((NP, LL), jnp.float32)
  zeros64 = jnp.zeros((NP, HH), jnp.float32)

  cnt_part = _sc_degree(dst3)
  g1, u1 = _tc_prep1(x, W1, b1, cnt_part)
  s1_part = _sc_msgpass(g1, src3, dst3, zeros64)
  pqw, dinv_p, us_p, ut_p = _tc_mid(s1_part, u1, W2, b2, Wfc, bfc, cnt_part)
  dec = _sc_scalar_decode(pqw, src3, dst3, zeros2, dinv_p, us_p, ut_p, ei3)
  return dec.reshape(EE, 1)


# speedup vs baseline: 1.2945x; 1.0218x over previous
import functools

import jax
import jax.numpy as jnp
from jax import lax
from jax.experimental import pallas as pl
from jax.experimental.pallas import tpu as pltpu
from jax.experimental.pallas import tpu_sc as plsc

NN = 10000
EE = 320000
DD = 128
HH = 64
NC, NS, LL = 2, 16, 16
NWORK = NC * NS
CHUNK = 128
NCH = 80
NBUF = 2
NGRP = NCH // NBUF
EPAD = NWORK * NCH * CHUNK
ROWS_PER_TILE = 632
NP = NS * ROWS_PER_TILE
EW = EE // NWORK

_MESH = plsc.VectorSubcoreMesh(
    core_axis_name="c", subcore_axis_name="s", num_cores=NC, num_subcores=NS)


def _wid():
  return lax.axis_index("c") * NS + lax.axis_index("s")


@functools.partial(
    pl.kernel,
    out_type=jax.ShapeDtypeStruct((NC, NP), jnp.float32),
    mesh=_MESH,
    compiler_params=pltpu.CompilerParams(
        use_tc_tiling_on_sc=False, needs_layout_passes=False),
    scratch_types=[
        pltpu.VMEM((NCH, CHUNK), jnp.int32),
        pltpu.VMEM((NP,), jnp.float32),
        pltpu.VMEM((ROWS_PER_TILE,), jnp.float32),
        pltpu.VMEM((ROWS_PER_TILE,), jnp.float32),
        pltpu.VMEM_SHARED((NS, NP), jnp.float32),
    ],
)
def _sc_degree(dst3, cnt_out, dst_l, cnt_l, accb, tbuf, stage):
  c = lax.axis_index("c")
  s = lax.axis_index("s")
  w = _wid()
  rbase = s * ROWS_PER_TILE
  rpt = pl.ds(rbase, ROWS_PER_TILE)
  pltpu.sync_copy(dst3.at[w], dst_l)

  @pl.loop(0, NP // LL)
  def _(i):
    cnt_l[pl.ds(i * LL, LL)] = jnp.zeros((LL,), jnp.float32)

  ones = jnp.ones((LL,), jnp.float32)

  @pl.loop(0, NCH)
  def _(j):
    for k in range(CHUNK // LL):
      dv = dst_l[j, pl.ds(k * LL, LL)]
      plsc.addupdate_scatter(cnt_l, [dv], ones)

  pltpu.sync_copy(cnt_l, stage.at[s])
  plsc.subcore_barrier()

  @pl.loop(0, ROWS_PER_TILE // LL)
  def _(i):
    accb[pl.ds(i * LL, LL)] = jnp.zeros((LL,), jnp.float32)

  for t in range(NS):
    pltpu.sync_copy(stage.at[t, rpt], tbuf)

    @pl.loop(0, ROWS_PER_TILE // LL)
    def _(i):
      sl = pl.ds(i * LL, LL)
      accb[sl] = accb[sl] + tbuf[sl]

  pltpu.sync_copy(accb, cnt_out.at[c, rpt])


@functools.partial(
    pl.kernel,
    out_type=jax.ShapeDtypeStruct((NC, NP, HH), jnp.float32),
    mesh=_MESH,
    compiler_params=pltpu.CompilerParams(use_tc_tiling_on_sc=False),
    scratch_types=[
        pltpu.VMEM((NCH, CHUNK), jnp.int32),
        pltpu.VMEM((NCH, CHUNK), jnp.int32),
        [pltpu.VMEM((CHUNK, HH), jnp.float32) for _ in range(NBUF)],
        [pltpu.SemaphoreType.DMA for _ in range(NBUF)],
        [pltpu.SemaphoreType.DMA for _ in range(NBUF)],
        pltpu.VMEM_SHARED((NP, HH), jnp.float32),
        pltpu.VMEM_SHARED((NP, HH), jnp.float32),
    ],
)
def _sc_msgpass(g_tab, src3, dst3, zeros64, s_out,
                src_l, dst_l, rows, gsem, ssem, acc, g_sp):
  c = lax.axis_index("c")
  s = lax.axis_index("s")
  w = _wid()
  rbase = s * ROWS_PER_TILE
  pltpu.sync_copy(g_tab.at[pl.ds(rbase, ROWS_PER_TILE)],
                  g_sp.at[pl.ds(rbase, ROWS_PER_TILE)])
  pltpu.sync_copy(zeros64.at[pl.ds(rbase, ROWS_PER_TILE)],
                  acc.at[pl.ds(rbase, ROWS_PER_TILE)])
  pltpu.sync_copy(src3.at[w], src_l)
  pltpu.sync_copy(dst3.at[w], dst_l)
  plsc.subcore_barrier()

  rows0, rows1 = rows[0], rows[1]
  sem0, sem1 = gsem[0], gsem[1]
  pltpu.async_copy(g_sp.at[src_l.at[0]], rows0, sem0)

  @pl.loop(0, NCH - 1)
  def _(j):
    even = j % 2 == 0

    def do(cur, nxt, sem_cur, sem_nxt):
      pltpu.async_copy(g_sp.at[src_l.at[j + 1]], nxt, sem_nxt)
      pltpu.make_async_copy(g_sp.at[src_l.at[j]], cur, sem_cur).wait()
      pltpu.sync_copy(cur, acc.at[dst_l.at[j]], add=True)

    @pl.when(even)
    def _():
      do(rows0, rows1, sem0, sem1)

    @pl.when(jnp.logical_not(even))
    def _():
      do(rows1, rows0, sem1, sem0)

  last = NCH - 1
  pltpu.make_async_copy(g_sp.at[src_l.at[last]], rows1, sem1).wait()
  pltpu.sync_copy(rows1, acc.at[dst_l.at[last]], add=True)

  plsc.subcore_barrier()
  pltpu.sync_copy(acc.at[pl.ds(rbase, ROWS_PER_TILE)],
                  s_out.at[c, pl.ds(rbase, ROWS_PER_TILE)])


@functools.partial(
    pl.kernel,
    out_type=jax.ShapeDtypeStruct((NWORK, NCH * CHUNK), jnp.float32),
    mesh=_MESH,
    compiler_params=pltpu.CompilerParams(
        use_tc_tiling_on_sc=False, needs_layout_passes=False),
    scratch_types=[
        pltpu.VMEM((2 * NCH, CHUNK), jnp.int32),
        pltpu.VMEM((2 * NCH, CHUNK), jnp.int32),
        pltpu.VMEM((NP,), jnp.float32),
        pltpu.VMEM((NP,), jnp.float32),
        pltpu.VMEM((NP,), jnp.float32),
        pltpu.VMEM((NP,), jnp.float32),
        pltpu.VMEM((640,), jnp.float32),
        pltpu.VMEM((640,), jnp.float32),
        pltpu.VMEM((640,), jnp.float32),
        pltpu.VMEM((640,), jnp.float32),
        pltpu.VMEM((640,), jnp.float32),
        pltpu.VMEM((640,), jnp.float32),
        pltpu.VMEM((NP,), jnp.float32),
        pltpu.VMEM((NP,), jnp.float32),
        pltpu.VMEM((NCH * CHUNK,), jnp.float32),
        pltpu.VMEM_SHARED((NS, NP), jnp.float32),
        pltpu.VMEM_SHARED((NP,), jnp.float32),
        pltpu.VMEM_SHARED((NP,), jnp.float32),
    ],
)
def _sc_scalar_decode(p_p, q_p, src3, dst3, dinv_p, us_p, ut_p,
                      dec_out,
                      src_l, dst_l, p_l, q_l, pacc, qacc,
                      dinv_l, us_l, ut_l, accb, tbuf, sv_l,
                      s_l, t_l, ob,
                      stage, s_sp, t_sp):
  c = lax.axis_index("c")
  s = lax.axis_index("s")
  rbase = s * ROWS_PER_TILE
  rpt = pl.ds(rbase, ROWS_PER_TILE)
  pltpu.sync_copy(p_p, p_l)
  pltpu.sync_copy(q_p, q_l)
  pltpu.sync_copy(src3.at[s], src_l.at[pl.ds(0, NCH)])
  pltpu.sync_copy(src3.at[s + NS], src_l.at[pl.ds(NCH, NCH)])
  pltpu.sync_copy(dst3.at[s], dst_l.at[pl.ds(0, NCH)])
  pltpu.sync_copy(dst3.at[s + NS], dst_l.at[pl.ds(NCH, NCH)])
  pltpu.sync_copy(dinv_p.at[rpt], dinv_l.at[pl.ds(0, ROWS_PER_TILE)])
  pltpu.sync_copy(us_p.at[rpt], us_l.at[pl.ds(0, ROWS_PER_TILE)])
  pltpu.sync_copy(ut_p.at[rpt], ut_l.at[pl.ds(0, ROWS_PER_TILE)])

  @pl.loop(0, NP // LL)
  def _(i):
    z = jnp.zeros((LL,), jnp.float32)
    pacc[pl.ds(i * LL, LL)] = z
    qacc[pl.ds(i * LL, LL)] = z

  @pl.loop(0, 2 * NCH)
  def _(j):
    for k in range(CHUNK // LL):
      sv = src_l[j, pl.ds(k * LL, LL)]
      dv = dst_l[j, pl.ds(k * LL, LL)]
      pv = plsc.load_gather(p_l, [sv])
      qv = plsc.load_gather(q_l, [sv])
      plsc.addupdate_scatter(pacc, [dv], pv)
      plsc.addupdate_scatter(qacc, [dv], qv)

  def reduce_into(ul, dst_sp):
    @pl.loop(0, 640 // LL)
    def _(i):
      accb[pl.ds(i * LL, LL)] = jnp.zeros((LL,), jnp.float32)

    for t in range(NS):
      pltpu.sync_copy(stage.at[t, rpt], tbuf.at[pl.ds(0, ROWS_PER_TILE)])

      @pl.loop(0, 640 // LL)
      def _(i):
        sl = pl.ds(i * LL, LL)
        accb[sl] = accb[sl] + tbuf[sl]

    @pl.loop(0, 640 // LL)
    def _(i):
      sl = pl.ds(i * LL, LL)
      sv_l[sl] = dinv_l[sl] * accb[sl] + ul[sl]

    pltpu.sync_copy(sv_l.at[pl.ds(0, ROWS_PER_TILE)], dst_sp.at[rpt])

  pltpu.sync_copy(pacc, stage.at[s])
  plsc.subcore_barrier()
  reduce_into(us_l, s_sp)
  plsc.subcore_barrier()
  pltpu.sync_copy(qacc, stage.at[s])
  plsc.subcore_barrier()
  reduce_into(ut_l, t_sp)
  plsc.subcore_barrier()

  pltpu.sync_copy(s_sp, s_l)
  pltpu.sync_copy(t_sp, t_l)
  w = _wid()
  wrow = c * NCH

  @pl.loop(0, NCH)
  def _(j):
    for k in range(CHUNK // LL):
      si = src_l[wrow + j, pl.ds(k * LL, LL)]
      di = dst_l[wrow + j, pl.ds(k * LL, LL)]
      sv = plsc.load_gather(s_l, [si])
      tv = plsc.load_gather(t_l, [di])
      y = sv + tv
      ob[pl.ds(j * CHUNK + k * LL, LL)] = 1.0 / (1.0 + jnp.exp(-y))

  pltpu.sync_copy(ob, dec_out.at[w])


def _dinv_from_cnt(cnt_ref):
  cnt = cnt_ref[0, 0:NN] + cnt_ref[1, 0:NN]
  return lax.rsqrt(cnt + 1.0)[:, None]


def _tc_prep1_body(x_ref, w1_ref, b1_ref, cnt_ref, g1_ref, u1_ref):
  dinv = _dinv_from_cnt(cnt_ref)
  h = jnp.dot(x_ref[...], w1_ref[...], preferred_element_type=jnp.float32)
  g1_ref[...] = jnp.concatenate(
      [dinv * h, jnp.zeros((NP - NN, HH), jnp.float32)], axis=0)
  u1_ref[...] = dinv * dinv * h + b1_ref[...]


def _tc_mid_body(sp_ref, u1_ref, w2_ref, b2_ref, wfc_ref, bfc_ref, cnt_ref,
                 p_ref, q_ref, dinv_ref, us_ref, ut_ref):
  dinv = _dinv_from_cnt(cnt_ref)
  ssum = sp_ref[0, 0:NN, :] + sp_ref[1, 0:NN, :]
  z1 = jnp.maximum(dinv * ssum + u1_ref[...], 0.0)
  h2 = jnp.dot(z1, w2_ref[...], preferred_element_type=jnp.float32)
  u2 = dinv * dinv * h2 + b2_ref[...]
  wa = wfc_ref[0:HH, 0]
  wb = wfc_ref[HH:2 * HH, 0]
  p = dinv[:, 0] * jnp.dot(h2, wa, preferred_element_type=jnp.float32)
  q = dinv[:, 0] * jnp.dot(h2, wb, preferred_element_type=jnp.float32)
  ztail = jnp.zeros((NP - NN,), jnp.float32)
  p_ref[...] = jnp.concatenate([p, ztail])
  q_ref[...] = jnp.concatenate([q, ztail])
  dinv_ref[...] = jnp.concatenate([dinv[:, 0], ztail])
  us_ref[...] = jnp.concatenate(
      [jnp.dot(u2, wa, preferred_element_type=jnp.float32) + bfc_ref[...],
       ztail])
  ut_ref[...] = jnp.concatenate(
      [jnp.dot(u2, wb, preferred_element_type=jnp.float32), ztail])


_tc_prep1 = pl.pallas_call(
    _tc_prep1_body,
    out_shape=[jax.ShapeDtypeStruct((NP, HH), jnp.float32),
               jax.ShapeDtypeStruct((NN, HH), jnp.float32)],
)

_tc_mid = pl.pallas_call(
    _tc_mid_body,
    out_shape=[jax.ShapeDtypeStruct((NP,), jnp.float32),
               jax.ShapeDtypeStruct((NP,), jnp.float32),
               jax.ShapeDtypeStruct((NP,), jnp.float32),
               jax.ShapeDtypeStruct((NP,), jnp.float32),
               jax.ShapeDtypeStruct((NP,), jnp.float32)],
)


def kernel(x, edge_index, W1, b1, W2, b2, Wfc, bfc):
  src = edge_index[0]
  dst = edge_index[1]
  npad = EPAD - EE
  srcp = jnp.concatenate([src, jnp.zeros((npad,), jnp.int32)])
  dstp = jnp.concatenate([dst, jnp.full((npad,), NN, jnp.int32)])
  src3 = srcp.reshape(NWORK, NCH, CHUNK)
  dst3 = dstp.reshape(NWORK, NCH, CHUNK)

  zeros64 = jnp.zeros((NP, HH), jnp.float32)

  cnt_part = _sc_degree(dst3)
  g1, u1 = _tc_prep1(x, W1, b1, cnt_part)
  s1_part = _sc_msgpass(g1, src3, dst3, zeros64)
  p_p, q_p, dinv_p, us_p, ut_p = _tc_mid(s1_part, u1, W2, b2, Wfc, bfc,
                                          cnt_part)
  dec = _sc_scalar_decode(p_p, q_p, src3, dst3, dinv_p, us_p, ut_p)
  return dec.reshape(EPAD)[:EE].reshape(EE, 1)
